# baseline (speedup 1.0000x reference)
import functools

import jax
import jax.numpy as jnp
from jax import lax
from jax.experimental import pallas as pl
from jax.experimental.pallas import tpu as pltpu
from jax.experimental.pallas import tpu_sc as plsc

VOCAB = 1000000
DIM = 64
PDIM = 2 * DIM
NPAIR = VOCAB // 2
N = 16384 * 20

NC = 2
NS = 16
L = 16
NW = NC * NS

RCH = 320
RNCH = VOCAB // RCH
RIT = -(-RNCH // NW)

BPW = N // NW
C = 160
NBUF = 2
NCH = BPW // C
assert NCH % NBUF == 0
assert (RIT - NBUF) % NBUF == 0

_mesh = plsc.VectorSubcoreMesh(core_axis_name="c", subcore_axis_name="s")
_params = pltpu.CompilerParams(needs_layout_passes=False)


@functools.partial(
    pl.kernel,
    mesh=_mesh,
    compiler_params=_params,
    out_type=jax.ShapeDtypeStruct((NPAIR, PDIM), jnp.float32),
    scratch_types=[
        pltpu.VMEM((NBUF, RCH // 8, 8, DIM), jnp.float32),
        pltpu.VMEM((NBUF, RCH // 2, PDIM), jnp.float32),
        pltpu.SemaphoreType.DMA,
        pltpu.SemaphoreType.DMA,
        pltpu.SemaphoreType.DMA,
        pltpu.SemaphoreType.DMA,
    ],
)
def _pack(table_hbm, packed_hbm, ina_v, pair_v, r0s, r1s, w0s, w1s):
    rsems = (r0s, r1s)
    wsems = (w0s, w1s)
    wid = lax.axis_index("s") * NC + lax.axis_index("c")

    def start_read(i, b):
        c = i * NW + wid

        @pl.when(c < RNCH)
        def _():
            pltpu.async_copy(
                table_hbm.at[pl.ds(c * (RCH // 8), RCH // 8), :, :],
                ina_v.at[b],
                rsems[b],
            )

    def start_write(i, b):
        c = i * NW + wid

        @pl.when(c < RNCH)
        def _():
            pltpu.async_copy(
                pair_v.at[b],
                packed_hbm.at[pl.ds(c * (RCH // 2), RCH // 2), :],
                wsems[b],
            )

    def wait_write(i, b):
        c = i * NW + wid

        @pl.when((c < RNCH) & (i >= 0))
        def _():
            pltpu.make_async_copy(
                pair_v.at[b],
                packed_hbm.at[pl.ds(c * (RCH // 2), RCH // 2), :],
                wsems[b],
            ).wait()

    def drain(i, b):
        c = i * NW + wid

        @pl.when(c < RNCH)
        def _():
            pltpu.make_async_copy(
                table_hbm.at[pl.ds(c * (RCH // 8), RCH // 8), :, :],
                ina_v.at[b],
                rsems[b],
            ).wait()
            wait_write(i - NBUF, b)

            def grp_body(grp, carry):
                for lq in range(0, L, 4):
                    vals = []
                    for l in range(lq, lq + 4):
                        rr = grp * 2 + (l >> 3)
                        s = l & 7
                        for cg in range(DIM // L):
                            vals.append(ina_v[b, rr, s, pl.ds(cg * L, L)])
                    vi = 0
                    for l in range(lq, lq + 4):
                        j = grp * (L // 2) + (l >> 1)
                        off = (l & 1) * DIM
                        for cg in range(DIM // L):
                            pair_v[b, j, pl.ds(off + cg * L, L)] = vals[vi]
                            vi += 1
                return carry

            lax.fori_loop(0, RCH // L, grp_body, 0)

    start_read(0, 0)
    start_read(1, 1)

    def steady(g0, carry):
        for b in range(NBUF):
            i = g0 * NBUF + b
            drain(i, b)
            start_write(i, b)
            start_read(i + NBUF, b)
        return carry

    lax.fori_loop(0, (RIT - NBUF) // NBUF, steady, 0)
    for i in range(RIT - NBUF, RIT):
        drain(i, i % NBUF)
        start_write(i, i % NBUF)
        wait_write(i, i % NBUF)


@functools.partial(
    pl.kernel,
    mesh=_mesh,
    compiler_params=_params,
    out_type=jax.ShapeDtypeStruct((N // 8, 8, DIM), jnp.float32),
    scratch_types=[
        pltpu.VMEM((BPW,), jnp.int32),
        pltpu.VMEM((BPW,), jnp.int32),
        pltpu.VMEM((NBUF, C, PDIM), jnp.float32),
        pltpu.VMEM((NBUF, C // 8, 8, DIM), jnp.float32),
        pltpu.SemaphoreType.DMA,
        pltpu.SemaphoreType.DMA,
        pltpu.SemaphoreType.DMA,
        pltpu.SemaphoreType.DMA,
    ],
)
def _gather(
    idx_hbm, packed_hbm, out_hbm, half_v, pair_v, rows_v, obuf_v, g0s, g1s, o0s, o1s
):
    gsems = (g0s, g1s)
    osems = (o0s, o1s)
    wid = lax.axis_index("s") * NC + lax.axis_index("c")
    base = wid * BPW

    pltpu.sync_copy(idx_hbm.at[pl.ds(base, BPW)], half_v)

    vocab = jnp.full((L,), VOCAB, jnp.int32)
    one = jnp.full((L,), 1, jnp.int32)

    def split_body(i, carry):
        s = pl.ds(i * L, L)
        idx = lax.rem(half_v[s], vocab)
        pair_v[s] = lax.shift_right_logical(idx, 1)
        half_v[s] = lax.bitwise_and(idx, one)
        return carry

    lax.fori_loop(0, BPW // L, split_body, 0)

    def start_gather(g, b):
        pltpu.async_copy(
            packed_hbm.at[pair_v.at[pl.ds(g * C, C)]], rows_v.at[b], gsems[b]
        )

    def wait_gather(g, b):
        pltpu.make_async_copy(
            packed_hbm.at[pair_v.at[pl.ds(g * C, C)]], rows_v.at[b], gsems[b]
        ).wait()

    def start_out(g, b):
        pltpu.async_copy(
            obuf_v.at[b],
            out_hbm.at[pl.ds((base + g * C) // 8, C // 8), :, :],
            osems[b],
        )

    def wait_out(g, b):
        @pl.when(g >= 0)
        def _():
            pltpu.make_async_copy(
                obuf_v.at[b],
                out_hbm.at[pl.ds((base + g * C) // 8, C // 8), :, :],
                osems[b],
            ).wait()

    def select(g, b):
        def grp16(k, carry):
            hvec = half_v[pl.ds(g * C + k * L, L)]
            for lq in range(0, L, 4):
                vals = []
                for l in range(lq, lq + 4):
                    off = lax.mul(hvec[l], DIM)
                    r = k * L + l
                    for cg in range(DIM // L):
                        vals.append(rows_v[b, r, pl.ds(off + cg * L, L)])
                vi = 0
                for l in range(lq, lq + 4):
                    rr = k * 2 + (l >> 3)
                    s = l & 7
                    for cg in range(DIM // L):
                        obuf_v[b, rr, s, pl.ds(cg * L, L)] = vals[vi]
                        vi += 1
            return carry

        lax.fori_loop(0, C // L, grp16, 0)

    for b in range(NBUF):
        start_gather(b, b)

    def steady(g0, carry):
        for b in range(NBUF):
            g = g0 * NBUF + b
            wait_gather(g, b)
            wait_out(g - NBUF, b)
            select(g, b)
            start_gather(g + NBUF, b)
            start_out(g, b)
        return carry

    lax.fori_loop(0, (NCH - NBUF) // NBUF, steady, 0)

    for b in range(NBUF):
        g = NCH - NBUF + b
        wait_gather(g, b)
        wait_out(g - NBUF, b)
        select(g, b)
        start_out(g, b)
        wait_out(g, b)


def kernel(indices, table):
    packed = jnp.reshape(table, (NPAIR, PDIM))
    out3 = _gather(indices.astype(jnp.int32), packed)
    return jnp.reshape(out3, (N, DIM))


# speedup vs baseline: 1.0847x; 1.0847x over previous
import functools

import jax
import jax.numpy as jnp
from jax import lax
from jax.experimental import pallas as pl
from jax.experimental.pallas import tpu as pltpu
from jax.experimental.pallas import tpu_sc as plsc

VOCAB = 1000000
DIM = 64
PDIM = 2 * DIM
NPAIR = VOCAB // 2
N = 16384 * 20

NC = 2
NS = 16
L = 16
NW = NC * NS

RCH = 320
RNCH = VOCAB // RCH
RIT = -(-RNCH // NW)

BPW = N // NW
C = 160
NBUF = 2
NCH = BPW // C
assert NCH % NBUF == 0
assert (RIT - NBUF) % NBUF == 0

_mesh = plsc.VectorSubcoreMesh(core_axis_name="c", subcore_axis_name="s")
_params = pltpu.CompilerParams(needs_layout_passes=False)


@functools.partial(
    pl.kernel,
    mesh=_mesh,
    compiler_params=_params,
    out_type=jax.ShapeDtypeStruct((NPAIR, PDIM), jnp.float32),
    scratch_types=[
        pltpu.VMEM((NBUF, RCH // 8, 8, DIM), jnp.float32),
        pltpu.VMEM((NBUF, RCH // 2, PDIM), jnp.float32),
        pltpu.SemaphoreType.DMA,
        pltpu.SemaphoreType.DMA,
        pltpu.SemaphoreType.DMA,
        pltpu.SemaphoreType.DMA,
    ],
)
def _pack(table_hbm, packed_hbm, ina_v, pair_v, r0s, r1s, w0s, w1s):
    rsems = (r0s, r1s)
    wsems = (w0s, w1s)
    wid = lax.axis_index("s") * NC + lax.axis_index("c")

    def start_read(i, b):
        c = i * NW + wid

        @pl.when(c < RNCH)
        def _():
            pltpu.async_copy(
                table_hbm.at[pl.ds(c * (RCH // 8), RCH // 8), :, :],
                ina_v.at[b],
                rsems[b],
            )

    def start_write(i, b):
        c = i * NW + wid

        @pl.when(c < RNCH)
        def _():
            pltpu.async_copy(
                pair_v.at[b],
                packed_hbm.at[pl.ds(c * (RCH // 2), RCH // 2), :],
                wsems[b],
            )

    def wait_write(i, b):
        c = i * NW + wid

        @pl.when((c < RNCH) & (i >= 0))
        def _():
            pltpu.make_async_copy(
                pair_v.at[b],
                packed_hbm.at[pl.ds(c * (RCH // 2), RCH // 2), :],
                wsems[b],
            ).wait()

    def drain(i, b):
        c = i * NW + wid

        @pl.when(c < RNCH)
        def _():
            pltpu.make_async_copy(
                table_hbm.at[pl.ds(c * (RCH // 8), RCH // 8), :, :],
                ina_v.at[b],
                rsems[b],
            ).wait()
            wait_write(i - NBUF, b)

            def grp_body(grp, carry):
                for lq in range(0, L, 4):
                    vals = []
                    for l in range(lq, lq + 4):
                        rr = grp * 2 + (l >> 3)
                        s = l & 7
                        for cg in range(DIM // L):
                            vals.append(ina_v[b, rr, s, pl.ds(cg * L, L)])
                    vi = 0
                    for l in range(lq, lq + 4):
                        j = grp * (L // 2) + (l >> 1)
                        off = (l & 1) * DIM
                        for cg in range(DIM // L):
                            pair_v[b, j, pl.ds(off + cg * L, L)] = vals[vi]
                            vi += 1
                return carry

            lax.fori_loop(0, RCH // L, grp_body, 0)

    start_read(0, 0)
    start_read(1, 1)

    def steady(g0, carry):
        for b in range(NBUF):
            i = g0 * NBUF + b
            drain(i, b)
            start_write(i, b)
            start_read(i + NBUF, b)
        return carry

    lax.fori_loop(0, (RIT - NBUF) // NBUF, steady, 0)
    for i in range(RIT - NBUF, RIT):
        drain(i, i % NBUF)
        start_write(i, i % NBUF)
        wait_write(i, i % NBUF)


@functools.partial(
    pl.kernel,
    mesh=_mesh,
    compiler_params=_params,
    out_type=jax.ShapeDtypeStruct((N // 8, 8, DIM), jnp.float32),
    scratch_types=[
        pltpu.VMEM((BPW,), jnp.int32),
        pltpu.VMEM((BPW,), jnp.int32),
        pltpu.VMEM((NBUF, C, PDIM), jnp.float32),
        pltpu.VMEM((NBUF, C // 8, 8, DIM), jnp.float32),
        pltpu.SemaphoreType.DMA,
        pltpu.SemaphoreType.DMA,
        pltpu.SemaphoreType.DMA,
        pltpu.SemaphoreType.DMA,
    ],
)
def _gather(
    idx_hbm, packed_hbm, out_hbm, half_v, pair_v, rows_v, obuf_v, g0s, g1s, o0s, o1s
):
    gsems = (g0s, g1s)
    osems = (o0s, o1s)
    wid = lax.axis_index("s") * NC + lax.axis_index("c")
    base = wid * BPW

    pltpu.sync_copy(idx_hbm.at[pl.ds(base, BPW)], half_v)

    vocab = jnp.full((L,), 2 * VOCAB, jnp.int32)
    one = jnp.full((L,), 1, jnp.int32)

    def split_body(i, carry):
        s = pl.ds(i * L, L)
        idx = lax.rem(half_v[s], vocab)
        pair_v[s] = lax.shift_right_logical(idx, 1)
        half_v[s] = lax.bitwise_and(idx, one)
        return carry

    lax.fori_loop(0, BPW // L, split_body, 0)

    def start_gather(g, b):
        pltpu.async_copy(
            packed_hbm.at[pair_v.at[pl.ds(g * C, C)]], rows_v.at[b], gsems[b]
        )

    def wait_gather(g, b):
        pltpu.make_async_copy(
            packed_hbm.at[pair_v.at[pl.ds(g * C, C)]], rows_v.at[b], gsems[b]
        ).wait()

    def start_out(g, b):
        pltpu.async_copy(
            obuf_v.at[b],
            out_hbm.at[pl.ds((base + g * C) // 8, C // 8), :, :],
            osems[b],
        )

    def wait_out(g, b):
        @pl.when(g >= 0)
        def _():
            pltpu.make_async_copy(
                obuf_v.at[b],
                out_hbm.at[pl.ds((base + g * C) // 8, C // 8), :, :],
                osems[b],
            ).wait()

    def select(g, b):
        def grp16(k, carry):
            hvec = half_v[pl.ds(g * C + k * L, L)]
            for lq in range(0, L, 4):
                vals = []
                for l in range(lq, lq + 4):
                    off = lax.mul(hvec[l], DIM)
                    r = k * L + l
                    for cg in range(DIM // L):
                        vals.append(rows_v[b, r, pl.ds(off + cg * L, L)])
                vi = 0
                for l in range(lq, lq + 4):
                    rr = k * 2 + (l >> 3)
                    s = l & 7
                    for cg in range(DIM // L):
                        obuf_v[b, rr, s, pl.ds(cg * L, L)] = vals[vi]
                        vi += 1
            return carry

        lax.fori_loop(0, C // L, grp16, 0)

    for b in range(NBUF):
        start_gather(b, b)

    def steady(g0, carry):
        for b in range(NBUF):
            g = g0 * NBUF + b
            wait_gather(g, b)
            wait_out(g - NBUF, b)
            select(g, b)
            start_gather(g + NBUF, b)
            start_out(g, b)
        return carry

    lax.fori_loop(0, (NCH - NBUF) // NBUF, steady, 0)

    for b in range(NBUF):
        g = NCH - NBUF + b
        wait_gather(g, b)
        wait_out(g - NBUF, b)
        select(g, b)
        start_out(g, b)
        wait_out(g, b)


def kernel(indices, table):
    padded = jnp.pad(table, ((0, 0), (0, DIM)))
    idx = indices.astype(jnp.int32)
    out3 = _gather(idx * 2, padded)
    return jnp.reshape(out3, (N, DIM))


# speedup vs baseline: 1.1656x; 1.0746x over previous
import functools

import jax
import jax.numpy as jnp
from jax import lax
from jax.experimental import pallas as pl
from jax.experimental.pallas import tpu as pltpu
from jax.experimental.pallas import tpu_sc as plsc

VOCAB = 1000000
DIM = 64
PDIM = 2 * DIM
NPAIR = VOCAB // 2
N = 16384 * 20

NC = 2
NS = 16
L = 16
NW = NC * NS

RCH = 320
RNCH = VOCAB // RCH
RIT = -(-RNCH // NW)

BPW = N // NW
C = 160
NBUF = 2
NCH = BPW // C
assert NCH % NBUF == 0
assert (RIT - NBUF) % NBUF == 0

_mesh = plsc.VectorSubcoreMesh(core_axis_name="c", subcore_axis_name="s")
_params = pltpu.CompilerParams(needs_layout_passes=False)


@functools.partial(
    pl.kernel,
    mesh=_mesh,
    compiler_params=_params,
    out_type=jax.ShapeDtypeStruct((NPAIR, PDIM), jnp.float32),
    scratch_types=[
        pltpu.VMEM((NBUF, RCH // 8, 8, DIM), jnp.float32),
        pltpu.VMEM((NBUF, RCH // 2, PDIM), jnp.float32),
        pltpu.SemaphoreType.DMA,
        pltpu.SemaphoreType.DMA,
        pltpu.SemaphoreType.DMA,
        pltpu.SemaphoreType.DMA,
    ],
)
def _pack(table_hbm, packed_hbm, ina_v, pair_v, r0s, r1s, w0s, w1s):
    rsems = (r0s, r1s)
    wsems = (w0s, w1s)
    wid = lax.axis_index("s") * NC + lax.axis_index("c")

    def start_read(i, b):
        c = i * NW + wid

        @pl.when(c < RNCH)
        def _():
            pltpu.async_copy(
                table_hbm.at[pl.ds(c * (RCH // 8), RCH // 8), :, :],
                ina_v.at[b],
                rsems[b],
            )

    def start_write(i, b):
        c = i * NW + wid

        @pl.when(c < RNCH)
        def _():
            pltpu.async_copy(
                pair_v.at[b],
                packed_hbm.at[pl.ds(c * (RCH // 2), RCH // 2), :],
                wsems[b],
            )

    def wait_write(i, b):
        c = i * NW + wid

        @pl.when((c < RNCH) & (i >= 0))
        def _():
            pltpu.make_async_copy(
                pair_v.at[b],
                packed_hbm.at[pl.ds(c * (RCH // 2), RCH // 2), :],
                wsems[b],
            ).wait()

    def drain(i, b):
        c = i * NW + wid

        @pl.when(c < RNCH)
        def _():
            pltpu.make_async_copy(
                table_hbm.at[pl.ds(c * (RCH // 8), RCH // 8), :, :],
                ina_v.at[b],
                rsems[b],
            ).wait()
            wait_write(i - NBUF, b)

            def grp_body(grp, carry):
                for lq in range(0, L, 8):
                    vals = []
                    for l in range(lq, lq + 8):
                        rr = grp * 2 + (l >> 3)
                        s = l & 7
                        for cg in range(DIM // L):
                            vals.append(ina_v[b, rr, s, pl.ds(cg * L, L)])
                    vi = 0
                    for l in range(lq, lq + 8):
                        j = grp * (L // 2) + (l >> 1)
                        off = (l & 1) * DIM
                        for cg in range(DIM // L):
                            pair_v[b, j, pl.ds(off + cg * L, L)] = vals[vi]
                            vi += 1
                return carry

            lax.fori_loop(0, RCH // L, grp_body, 0)

    start_read(0, 0)
    start_read(1, 1)

    def steady(g0, carry):
        for b in range(NBUF):
            i = g0 * NBUF + b
            drain(i, b)
            start_write(i, b)
            start_read(i + NBUF, b)
        return carry

    lax.fori_loop(0, (RIT - NBUF) // NBUF, steady, 0)
    for i in range(RIT - NBUF, RIT):
        drain(i, i % NBUF)
        start_write(i, i % NBUF)
        wait_write(i, i % NBUF)


@functools.partial(
    pl.kernel,
    mesh=_mesh,
    compiler_params=_params,
    out_type=jax.ShapeDtypeStruct((N // 8, 8, DIM), jnp.float32),
    scratch_types=[
        pltpu.VMEM((BPW,), jnp.int32),
        pltpu.VMEM((BPW,), jnp.int32),
        pltpu.VMEM((NBUF, C, PDIM), jnp.float32),
        pltpu.VMEM((NBUF, C // 8, 8, DIM), jnp.float32),
        pltpu.SemaphoreType.DMA,
        pltpu.SemaphoreType.DMA,
        pltpu.SemaphoreType.DMA,
        pltpu.SemaphoreType.DMA,
    ],
)
def _gather(
    idx_hbm, packed_hbm, out_hbm, half_v, pair_v, rows_v, obuf_v, g0s, g1s, o0s, o1s
):
    gsems = (g0s, g1s)
    osems = (o0s, o1s)
    wid = lax.axis_index("s") * NC + lax.axis_index("c")
    base = wid * BPW

    pltpu.sync_copy(idx_hbm.at[pl.ds(base, BPW)], half_v)

    vocab = jnp.full((L,), VOCAB, jnp.int32)
    one = jnp.full((L,), 1, jnp.int32)

    def split_body(i, carry):
        s = pl.ds(i * L, L)
        idx = lax.rem(half_v[s], vocab)
        pair_v[s] = lax.shift_right_logical(idx, 1)
        half_v[s] = lax.bitwise_and(idx, one)
        return carry

    lax.fori_loop(0, BPW // L, split_body, 0)

    def start_gather(g, b):
        pltpu.async_copy(
            packed_hbm.at[pair_v.at[pl.ds(g * C, C)]], rows_v.at[b], gsems[b]
        )

    def wait_gather(g, b):
        pltpu.make_async_copy(
            packed_hbm.at[pair_v.at[pl.ds(g * C, C)]], rows_v.at[b], gsems[b]
        ).wait()

    def start_out(g, b):
        pltpu.async_copy(
            obuf_v.at[b],
            out_hbm.at[pl.ds((base + g * C) // 8, C // 8), :, :],
            osems[b],
        )

    def wait_out(g, b):
        @pl.when(g >= 0)
        def _():
            pltpu.make_async_copy(
                obuf_v.at[b],
                out_hbm.at[pl.ds((base + g * C) // 8, C // 8), :, :],
                osems[b],
            ).wait()

    def select(g, b):
        def grp16(k, carry):
            hvec = half_v[pl.ds(g * C + k * L, L)]
            for lq in range(0, L, 8):
                vals = []
                for l in range(lq, lq + 8):
                    off = lax.mul(hvec[l], DIM)
                    r = k * L + l
                    for cg in range(DIM // L):
                        vals.append(rows_v[b, r, pl.ds(off + cg * L, L)])
                vi = 0
                for l in range(lq, lq + 8):
                    rr = k * 2 + (l >> 3)
                    s = l & 7
                    for cg in range(DIM // L):
                        obuf_v[b, rr, s, pl.ds(cg * L, L)] = vals[vi]
                        vi += 1
            return carry

        lax.fori_loop(0, C // L, grp16, 0)

    for b in range(NBUF):
        start_gather(b, b)

    def steady(g0, carry):
        for b in range(NBUF):
            g = g0 * NBUF + b
            wait_gather(g, b)
            wait_out(g - NBUF, b)
            select(g, b)
            start_gather(g + NBUF, b)
            start_out(g, b)
        return carry

    lax.fori_loop(0, (NCH - NBUF) // NBUF, steady, 0)

    for b in range(NBUF):
        g = NCH - NBUF + b
        wait_gather(g, b)
        wait_out(g - NBUF, b)
        select(g, b)
        start_out(g, b)
        wait_out(g, b)


def kernel(indices, table):
    table_g = jnp.reshape(table, (VOCAB // 8, 8, DIM))
    packed = _pack(table_g)
    out3 = _gather(indices.astype(jnp.int32), packed)
    return jnp.reshape(out3, (N, DIM))
